# baseline (speedup 1.0000x reference)
import jax
import jax.numpy as jnp
from jax import lax
from jax.experimental import pallas as pl
from jax.experimental.pallas import tpu as pltpu
from jax.experimental.pallas import tpu_sc as plsc

F = 26
B = 4096
MK = 1024
M = 4
PLEN = 16
N = B * F
NW = 32
NPW = N // NW
NE = NPW * M
OUTCH = NPW // 2
NCHUNK = NE // OUTCH
DMA_I = 128
S1_DMAS = NE // DMA_I
S2_DMAS = OUTCH // DMA_I
PER = 208
PV = PER // 16


def _vperm(vec, idx):
    dnums = lax.GatherDimensionNumbers(
        offset_dims=(), collapsed_slice_dims=(0,), start_index_map=(0,))
    return lax.gather(vec, idx[:, None], dnums, (1,),
                      mode=lax.GatherScatterMode.PROMISE_IN_BOUNDS)


def _body(x_hbm, cbi_raw, cb2_raw, off1_hbm, off2_hbm, out_hbm,
          xv, idx1, cr, gv, outv, o1v, o2v, sem1, sem2):
    nc = 2
    wid = lax.axis_index("s") * nc + lax.axis_index("c")
    base = wid * NPW
    cbi_hbm = cbi_raw
    cb2_hbm = cb2_raw

    pltpu.sync_copy(off1_hbm, o1v)
    pltpu.sync_copy(off2_hbm, o2v)
    pltpu.sync_copy(x_hbm.at[pl.ds(base, NPW)], xv)

    lane = lax.iota(jnp.int32, 16)
    rep4 = lax.shift_right_logical(lane, 2)

    def mk_idx1(t, _):
        xvec = xv[pl.ds(t * 16, 16)]
        for j in range(4):
            x4 = _vperm(xvec, rep4 + 4 * j)
            q = 4 * t + j
            p = 16 * lax.rem(q, PV)
            idx1[pl.ds(q * 16, 16)] = (x4 << 2) + o1v[pl.ds(p, 16)]
        return _
    lax.fori_loop(0, NPW // 16, mk_idx1, None)

    def s1_start(j, _):
        pltpu.make_async_copy(
            cbi_hbm.at[idx1.at[pl.ds(j * DMA_I, DMA_I)]],
            cr.at[pl.ds(j * DMA_I, DMA_I)], sem1).start()
        return _
    def s1_wait(j, _):
        pltpu.make_async_copy(
            cbi_hbm.at[idx1.at[pl.ds(j * DMA_I, DMA_I)]],
            cr.at[pl.ds(j * DMA_I, DMA_I)], sem1).wait()
        return _
    lax.fori_loop(0, S1_DMAS, s1_start, None)
    lax.fori_loop(0, S1_DMAS, s1_wait, None)

    def mk_idx2(q, _):
        p = 16 * lax.rem(q, PV)
        cr[pl.ds(q * 16, 16)] = (cr[pl.ds(q * 16, 16)] << 2) + o2v[pl.ds(p, 16)]
        return _
    lax.fori_loop(0, NE // 16, mk_idx2, None)

    for k in range(NCHUNK):
        def s2_start(j, _):
            pltpu.make_async_copy(
                cb2_hbm.at[cr.at[pl.ds(k * OUTCH + j * DMA_I, DMA_I)]],
                gv.at[pl.ds(j * DMA_I, DMA_I)], sem2).start()
            return _
        def s2_wait(j, _):
            pltpu.make_async_copy(
                cb2_hbm.at[cr.at[pl.ds(k * OUTCH + j * DMA_I, DMA_I)]],
                gv.at[pl.ds(j * DMA_I, DMA_I)], sem2).wait()
            return _
        lax.fori_loop(0, S2_DMAS, s2_start, None)
        lax.fori_loop(0, S2_DMAS, s2_wait, None)

        def repack(r2, _):
            r = r2 * 8
            for jj in range(8):
                outv[r2, pl.ds(16 * jj, 16)] = gv[r + jj]
            return _
        lax.fori_loop(0, OUTCH // 8, repack, None)
        pltpu.sync_copy(
            outv, out_hbm.at[pl.ds(wid * (NE // 8) + k * (OUTCH // 8),
                                   OUTCH // 8)])


@jax.jit
def kernel(x, codebooks, cb_index):
    xf = x.reshape(N)
    ar = jnp.arange(PER, dtype=jnp.int32)
    off1 = ((ar // M) % F) * (4 * 40000) + (ar % M)
    off2 = ((ar // M) % F) * (MK * M) + (ar % M)

    run = pl.kernel(
        _body,
        out_type=jax.ShapeDtypeStruct((N * M * PLEN // 128, 128), jnp.float32),
        mesh=plsc.VectorSubcoreMesh(core_axis_name="c", subcore_axis_name="s"),
        scratch_types=[
            pltpu.VMEM((NPW,), jnp.int32),
            pltpu.VMEM((NE,), jnp.int32),
            pltpu.VMEM((NE,), jnp.int32),
            pltpu.VMEM((OUTCH, PLEN), jnp.float32),
            pltpu.VMEM((OUTCH // 8, 128), jnp.float32),
            pltpu.VMEM((PER,), jnp.int32),
            pltpu.VMEM((PER,), jnp.int32),
            pltpu.SemaphoreType.DMA,
            pltpu.SemaphoreType.DMA,
        ],
        compiler_params=pltpu.CompilerParams(use_tc_tiling_on_sc=False),
    )
    cbi = cb_index.reshape(cb_index.shape[0] * M)
    cb2 = codebooks.reshape(F * MK * M, PLEN)
    out = run(xf, cbi, cb2, off1, off2)
    return out.reshape(B, F, M * PLEN)


# speedup vs baseline: 7.2937x; 7.2937x over previous
import jax
import jax.numpy as jnp
from jax import lax
from jax.experimental import pallas as pl
from jax.experimental.pallas import tpu as pltpu
from jax.experimental.pallas import tpu_sc as plsc

F = 26
B = 4096
MK = 1024
M = 4
PLEN = 16
N = B * F
NW = 32
NPW = N // NW
NE = NPW * M
CH = NPW // 2
DMA_I = 128
S1_DMAS = NPW // DMA_I
S2_DMAS = CH // DMA_I
PER = 208
PV = PER // 16


def _body(x_hbm, cbiT_hbm, cb2_hbm, off1_hbm, off2_hbm, out_hbm,
          xv, cr4, idx2, widx, gv, o1v, o2v, sem1, sem2):
    nc = 2
    wid = lax.axis_index("s") * nc + lax.axis_index("c")
    base = wid * NPW

    pltpu.sync_copy(off1_hbm, o1v)
    pltpu.sync_copy(off2_hbm, o2v)
    pltpu.sync_copy(x_hbm.at[pl.ds(base, NPW)], xv)

    lane = lax.iota(jnp.int32, 16)

    def add_off(q, _):
        p = 16 * lax.rem(q, PV)
        xv[pl.ds(q * 16, 16)] = xv[pl.ds(q * 16, 16)] + o1v[pl.ds(p, 16)]
        return _
    lax.fori_loop(0, NPW // 16, add_off, None)

    def s1_start(j, _):
        for i in range(M):
            pltpu.make_async_copy(
                cbiT_hbm.at[i].at[xv.at[pl.ds(j * DMA_I, DMA_I)]],
                cr4.at[i, pl.ds(j * DMA_I, DMA_I)], sem1).start()
        return _
    def s1_wait(j, _):
        for i in range(M):
            pltpu.make_async_copy(
                cbiT_hbm.at[i].at[xv.at[pl.ds(j * DMA_I, DMA_I)]],
                cr4.at[i, pl.ds(j * DMA_I, DMA_I)], sem1).wait()
        return _
    lax.fori_loop(0, S1_DMAS, s1_start, None)
    lax.fori_loop(0, S1_DMAS, s1_wait, None)

    def mk_idx(q, _):
        p = 16 * lax.rem(q, PV)
        o2 = o2v[pl.ds(p, 16)]
        nvec = 4 * (base + 16 * q + lane)
        r = lax.shift_right_logical(q, 3)
        c = 16 * lax.rem(q, 8)
        for i in range(M):
            cvec = cr4[i, pl.ds(q * 16, 16)]
            idx2[pl.ds(i * NPW + q * 16, 16)] = (cvec << 2) + o2 + i
            widx[i * S1_DMAS + r, pl.ds(c, 16)] = nvec + i
        return _
    lax.fori_loop(0, NPW // 16, mk_idx, None)

    for i in range(M):
        for k in range(NPW // CH):
            def s2_start(j, _):
                pltpu.make_async_copy(
                    cb2_hbm.at[idx2.at[pl.ds(i * NPW + k * CH + j * DMA_I,
                                             DMA_I)]],
                    gv.at[pl.ds(j * DMA_I, DMA_I)], sem2).start()
                return _
            def s2_wait(j, _):
                pltpu.make_async_copy(
                    cb2_hbm.at[idx2.at[pl.ds(i * NPW + k * CH + j * DMA_I,
                                             DMA_I)]],
                    gv.at[pl.ds(j * DMA_I, DMA_I)], sem2).wait()
                return _
            lax.fori_loop(0, S2_DMAS, s2_start, None)
            lax.fori_loop(0, S2_DMAS, s2_wait, None)

            def s3_start(j, _):
                pltpu.make_async_copy(
                    gv.at[pl.ds(j * DMA_I, DMA_I)],
                    out_hbm.at[widx.at[i * S1_DMAS + k * S2_DMAS + j]],
                    sem1).start()
                return _
            def s3_wait(j, _):
                pltpu.make_async_copy(
                    gv.at[pl.ds(j * DMA_I, DMA_I)],
                    out_hbm.at[widx.at[i * S1_DMAS + k * S2_DMAS + j]],
                    sem1).wait()
                return _
            lax.fori_loop(0, S2_DMAS, s3_start, None)
            lax.fori_loop(0, S2_DMAS, s3_wait, None)


@jax.jit
def kernel(x, codebooks, cb_index):
    xf = x.reshape(N)
    cbiT = cb_index.T
    cb2 = codebooks.reshape(F * MK * M, PLEN)
    ar = jnp.arange(PER, dtype=jnp.int32)
    off1 = (ar % F) * 40000
    off2 = (ar % F) * (MK * M)

    run = pl.kernel(
        _body,
        out_type=jax.ShapeDtypeStruct((N * M, PLEN), jnp.float32),
        mesh=plsc.VectorSubcoreMesh(core_axis_name="c", subcore_axis_name="s"),
        scratch_types=[
            pltpu.VMEM((NPW,), jnp.int32),
            pltpu.VMEM((M, NPW), jnp.int32),
            pltpu.VMEM((M * NPW,), jnp.int32),
            pltpu.VMEM((M * S1_DMAS, DMA_I), jnp.int32),
            pltpu.VMEM((CH, PLEN), jnp.float32),
            pltpu.VMEM((PER,), jnp.int32),
            pltpu.VMEM((PER,), jnp.int32),
            pltpu.SemaphoreType.DMA,
            pltpu.SemaphoreType.DMA,
        ],
        compiler_params=pltpu.CompilerParams(use_tc_tiling_on_sc=False),
    )
    out = run(xf, cbiT, cb2, off1, off2)
    return out.reshape(B, F, M * PLEN)


# speedup vs baseline: 7.6884x; 1.0541x over previous
import jax
import jax.numpy as jnp
from jax import lax
from jax.experimental import pallas as pl
from jax.experimental.pallas import tpu as pltpu
from jax.experimental.pallas import tpu_sc as plsc

F = 26
B = 4096
MK = 1024
M = 4
PLEN = 16
N = B * F
NW = 32
NPW = N // NW
CH = NPW // 2
NCH = M * NPW // CH
DMA_I = 128
SC_DMAS = CH // DMA_I
WROWS = NPW // DMA_I


def _body(xT_hbm, cbiT_hbm, cb2_hbm, out_hbm,
          xv, cr4, idx2, widx, gv, sem1, sem2, sem3, sem4):
    nc = 2
    wid = lax.axis_index("s") * nc + lax.axis_index("c")
    base = wid * NPW

    pltpu.sync_copy(xT_hbm.at[pl.ds(base, NPW)], xv)

    lane = lax.iota(jnp.int32, 16)

    def add_off(q, _):
        nv = base + 16 * q + lane
        f = lax.shift_right_logical(nv, 12)
        xv[pl.ds(q * 16, 16)] = xv[pl.ds(q * 16, 16)] + f * 40000
        return _
    lax.fori_loop(0, NPW // 16, add_off, None)

    def s1(i):
        return pltpu.make_async_copy(cbiT_hbm.at[i].at[xv], cr4.at[i], sem1)
    for i in range(M):
        s1(i).start()

    def mk_widx(q, _):
        nv = base + 16 * q + lane
        b = nv & 4095
        f = lax.shift_right_logical(nv, 12)
        w0 = b * 104 + (f << 2)
        r = lax.shift_right_logical(q, 3)
        c = 16 * lax.rem(q, 8)
        for i in range(M):
            widx[i * WROWS + r, pl.ds(c, 16)] = w0 + i
        return _
    lax.fori_loop(0, NPW // 16, mk_widx, None)

    for i in range(M):
        s1(i).wait()

        def mk_idx(q, _, i=i):
            nv = base + 16 * q + lane
            cvec = cr4[i, pl.ds(q * 16, 16)]
            idx2[pl.ds(i * NPW + q * 16, 16)] = \
                (cvec << 2) + (nv & ~jnp.int32(4095)) + i
            return _
        lax.fori_loop(0, NPW // 16, mk_idx, None)

    def g(c):
        return pltpu.make_async_copy(
            cb2_hbm.at[idx2.at[pl.ds(c * CH, CH)]], gv.at[c & 1], sem2)

    def s(c, j, ssem):
        return pltpu.make_async_copy(
            gv.at[c & 1].at[pl.ds(j * DMA_I, DMA_I)],
            out_hbm.at[widx.at[c * SC_DMAS + j]], ssem)

    ssems = (sem3, sem4)
    g(0).start()
    for c in range(NCH):
        g(c).wait()

        def sc_start(j, _, c=c):
            s(c, j, ssems[c & 1]).start()
            return _
        lax.fori_loop(0, SC_DMAS, sc_start, None)
        if c + 1 < NCH:
            if c >= 1:
                def sc_wait(j, _, c=c):
                    s(c - 1, j, ssems[(c - 1) & 1]).wait()
                    return _
                lax.fori_loop(0, SC_DMAS, sc_wait, None)
            g(c + 1).start()
    for c in (NCH - 2, NCH - 1):
        def sc_wait2(j, _, c=c):
            s(c, j, ssems[c & 1]).wait()
            return _
        lax.fori_loop(0, SC_DMAS, sc_wait2, None)


@jax.jit
def kernel(x, codebooks, cb_index):
    xTf = x.T.reshape(N)
    cbiT = cb_index.T
    cb2 = codebooks.reshape(F * MK * M, PLEN)

    run = pl.kernel(
        _body,
        out_type=jax.ShapeDtypeStruct((N * M, PLEN), jnp.float32),
        mesh=plsc.VectorSubcoreMesh(core_axis_name="c", subcore_axis_name="s"),
        scratch_types=[
            pltpu.VMEM((NPW,), jnp.int32),
            pltpu.VMEM((M, NPW), jnp.int32),
            pltpu.VMEM((M * NPW,), jnp.int32),
            pltpu.VMEM((M * WROWS, DMA_I), jnp.int32),
            pltpu.VMEM((2, CH, PLEN), jnp.float32),
            pltpu.SemaphoreType.DMA,
            pltpu.SemaphoreType.DMA,
            pltpu.SemaphoreType.DMA,
            pltpu.SemaphoreType.DMA,
        ],
        compiler_params=pltpu.CompilerParams(use_tc_tiling_on_sc=False),
    )
    out = run(xTf, cbiT, cb2)
    return out.reshape(B, F, M * PLEN)


# speedup vs baseline: 9.3055x; 1.2103x over previous
import jax
import jax.numpy as jnp
from jax import lax
from jax.experimental import pallas as pl
from jax.experimental.pallas import tpu as pltpu
from jax.experimental.pallas import tpu_sc as plsc

F = 26
B = 4096
MK = 1024
M = 4
PLEN = 16
N = B * F
NW = 32
NPW = N // NW
CH = NPW // 2
NCH = M * NPW // CH
DMA_I = 128
SC_DMAS = CH // DMA_I
WROWS = NPW // DMA_I


def _body(xT_hbm, cbiT_hbm, cb2_hbm, out_hbm,
          xv, cr4, idx2, widx, gv, sem1, sem2, sem3, sem4):
    nc = 2
    wid = lax.axis_index("s") * nc + lax.axis_index("c")
    base = wid * NPW

    pltpu.sync_copy(xT_hbm.at[pl.ds(base, NPW)], xv)

    lane = lax.iota(jnp.int32, 16)

    def add_off(q, _):
        nv = base + 16 * q + lane
        f = lax.shift_right_logical(nv, 12)
        xv[pl.ds(q * 16, 16)] = xv[pl.ds(q * 16, 16)] + f * 40000
        return _
    lax.fori_loop(0, NPW // 16, add_off, None)

    def s1(i):
        return pltpu.make_async_copy(cbiT_hbm.at[i].at[xv], cr4.at[i], sem1)
    for i in range(M):
        s1(i).start()

    def mk_widx(q, _):
        nv = base + 16 * q + lane
        b = nv & 4095
        f = lax.shift_right_logical(nv, 12)
        w0 = (f << 14) + ((b & 2047) << 3) + \
            (lax.shift_right_logical(b, 11) << 2)
        r = lax.shift_right_logical(q, 3)
        c = 16 * lax.rem(q, 8)
        for i in range(M):
            widx[i * WROWS + r, pl.ds(c, 16)] = w0 + i
        return _
    lax.fori_loop(0, NPW // 16, mk_widx, None)

    for i in range(M):
        s1(i).wait()

        def mk_idx(q, _, i=i):
            nv = base + 16 * q + lane
            cvec = cr4[i, pl.ds(q * 16, 16)]
            idx2[pl.ds(i * NPW + q * 16, 16)] = \
                (cvec << 2) + (nv & ~jnp.int32(4095)) + i
            return _
        lax.fori_loop(0, NPW // 16, mk_idx, None)

    def g(c):
        return pltpu.make_async_copy(
            cb2_hbm.at[idx2.at[pl.ds(c * CH, CH)]], gv.at[c & 1], sem2)

    def s(c, j, ssem):
        return pltpu.make_async_copy(
            gv.at[c & 1].at[pl.ds(j * DMA_I, DMA_I)],
            out_hbm.at[widx.at[c * SC_DMAS + j]], ssem)

    ssems = (sem3, sem4)
    g(0).start()
    for c in range(NCH):
        g(c).wait()

        def sc_start(j, _, c=c):
            s(c, j, ssems[c & 1]).start()
            return _
        lax.fori_loop(0, SC_DMAS, sc_start, None)
        if c + 1 < NCH:
            if c >= 1:
                def sc_wait(j, _, c=c):
                    s(c - 1, j, ssems[(c - 1) & 1]).wait()
                    return _
                lax.fori_loop(0, SC_DMAS, sc_wait, None)
            g(c + 1).start()
    for c in (NCH - 2, NCH - 1):
        def sc_wait2(j, _, c=c):
            s(c, j, ssems[c & 1]).wait()
            return _
        lax.fori_loop(0, SC_DMAS, sc_wait2, None)


def _tc_transpose(in_ref, out_ref):
    out_ref[0, :, 0:B // 2] = in_ref[0, :, 0:M * PLEN].T
    out_ref[0, :, B // 2:B] = in_ref[0, :, M * PLEN:2 * M * PLEN].T


@jax.jit
def kernel(x, codebooks, cb_index):
    xTf = x.T.reshape(N)
    cbiT = cb_index.T
    cb2 = codebooks.reshape(F * MK * M, PLEN)

    run = pl.kernel(
        _body,
        out_type=jax.ShapeDtypeStruct((N * M, PLEN), jnp.float32),
        mesh=plsc.VectorSubcoreMesh(core_axis_name="c", subcore_axis_name="s"),
        scratch_types=[
            pltpu.VMEM((NPW,), jnp.int32),
            pltpu.VMEM((M, NPW), jnp.int32),
            pltpu.VMEM((M * NPW,), jnp.int32),
            pltpu.VMEM((M * WROWS, DMA_I), jnp.int32),
            pltpu.VMEM((2, CH, PLEN), jnp.float32),
            pltpu.SemaphoreType.DMA,
            pltpu.SemaphoreType.DMA,
            pltpu.SemaphoreType.DMA,
            pltpu.SemaphoreType.DMA,
        ],
        compiler_params=pltpu.CompilerParams(use_tc_tiling_on_sc=False),
    )
    out = run(xTf, cbiT, cb2)
    t = pl.pallas_call(
        _tc_transpose,
        grid=(F,),
        in_specs=[pl.BlockSpec((1, B // 2, 2 * M * PLEN),
                               lambda f: (f, 0, 0))],
        out_specs=pl.BlockSpec((1, M * PLEN, B), lambda f: (f, 0, 0)),
        out_shape=jax.ShapeDtypeStruct((F, M * PLEN, B), jnp.float32),
    )(out.reshape(F, B // 2, 2 * M * PLEN))
    return jnp.transpose(t, (2, 0, 1))


# speedup vs baseline: 9.4243x; 1.0128x over previous
import jax
import jax.numpy as jnp
from jax import lax
from jax.experimental import pallas as pl
from jax.experimental.pallas import tpu as pltpu
from jax.experimental.pallas import tpu_sc as plsc

F = 26
B = 4096
MK = 1024
M = 4
PLEN = 16
N = B * F
NW = 32
NPW = N // NW
CH = NPW // 2
NCH = M * NPW // CH
DMA_I = 128
SC_DMAS = CH // DMA_I
WROWS = NPW // DMA_I


def _body(xT_hbm, cbiT_hbm, cb2_hbm, out_hbm,
          xv, cr4, idx2, widx, gv, sem1, sem2, sem3, sem4):
    nc = 2
    wid = lax.axis_index("s") * nc + lax.axis_index("c")
    base = wid * NPW

    pltpu.sync_copy(xT_hbm.at[pl.ds(base, NPW)], xv)

    lane = lax.iota(jnp.int32, 16)

    def add_off(q, _):
        nv = base + 16 * q + lane
        f = lax.shift_right_logical(nv, 12)
        xv[pl.ds(q * 16, 16)] = xv[pl.ds(q * 16, 16)] + f * 40000
        return _
    lax.fori_loop(0, NPW // 16, add_off, None)

    def s1(i):
        return pltpu.make_async_copy(cbiT_hbm.at[i].at[xv], cr4.at[i], sem1)
    for i in range(M):
        s1(i).start()

    def mk_widx(q, _):
        nv = base + 16 * q + lane
        b = nv & 4095
        f = lax.shift_right_logical(nv, 12)
        w0 = (f << 14) + ((b & 2047) << 3) + \
            (lax.shift_right_logical(b, 11) << 2)
        r = lax.shift_right_logical(q, 3)
        c = 16 * lax.rem(q, 8)
        for i in range(M):
            widx[i * WROWS + r, pl.ds(c, 16)] = w0 + i
        return _
    lax.fori_loop(0, NPW // 16, mk_widx, None)

    def mk_idx(i):
        def step(q, _):
            nv = base + 16 * q + lane
            cvec = cr4[i, pl.ds(q * 16, 16)]
            idx2[pl.ds(i * NPW + q * 16, 16)] = \
                (cvec << 2) + (nv & ~jnp.int32(4095)) + i
            return _
        lax.fori_loop(0, NPW // 16, step, None)

    def g(c):
        return pltpu.make_async_copy(
            cb2_hbm.at[idx2.at[pl.ds(c * CH, CH)]], gv.at[c & 1], sem2)

    def s(c, j, ssem):
        return pltpu.make_async_copy(
            gv.at[c & 1].at[pl.ds(j * DMA_I, DMA_I)],
            out_hbm.at[widx.at[c * SC_DMAS + j]], ssem)

    ssems = (sem3, sem4)
    s1(0).wait()
    mk_idx(0)
    g(0).start()
    for c in range(NCH):
        if c + 1 < NCH and (c + 1) % 2 == 0:
            s1((c + 1) >> 1).wait()
            mk_idx((c + 1) >> 1)
        g(c).wait()

        def sc_start(j, _, c=c):
            s(c, j, ssems[c & 1]).start()
            return _
        lax.fori_loop(0, SC_DMAS, sc_start, None)
        if c + 1 < NCH:
            if c >= 1:
                def sc_wait(j, _, c=c):
                    s(c - 1, j, ssems[(c - 1) & 1]).wait()
                    return _
                lax.fori_loop(0, SC_DMAS, sc_wait, None)
            g(c + 1).start()
    for c in (NCH - 2, NCH - 1):
        def sc_wait2(j, _, c=c):
            s(c, j, ssems[c & 1]).wait()
            return _
        lax.fori_loop(0, SC_DMAS, sc_wait2, None)


def _tc_transpose(in_ref, out_ref):
    out_ref[0, :, 0:B // 2] = in_ref[0, :, 0:M * PLEN].T
    out_ref[0, :, B // 2:B] = in_ref[0, :, M * PLEN:2 * M * PLEN].T


@jax.jit
def kernel(x, codebooks, cb_index):
    xTf = x.T.reshape(N)
    cbiT = cb_index.T
    cb2 = codebooks.reshape(F * MK * M, PLEN)

    run = pl.kernel(
        _body,
        out_type=jax.ShapeDtypeStruct((N * M, PLEN), jnp.float32),
        mesh=plsc.VectorSubcoreMesh(core_axis_name="c", subcore_axis_name="s"),
        scratch_types=[
            pltpu.VMEM((NPW,), jnp.int32),
            pltpu.VMEM((M, NPW), jnp.int32),
            pltpu.VMEM((M * NPW,), jnp.int32),
            pltpu.VMEM((M * WROWS, DMA_I), jnp.int32),
            pltpu.VMEM((2, CH, PLEN), jnp.float32),
            pltpu.SemaphoreType.DMA,
            pltpu.SemaphoreType.DMA,
            pltpu.SemaphoreType.DMA,
            pltpu.SemaphoreType.DMA,
        ],
        compiler_params=pltpu.CompilerParams(use_tc_tiling_on_sc=False),
    )
    out = run(xTf, cbiT, cb2)
    t = pl.pallas_call(
        _tc_transpose,
        grid=(F,),
        in_specs=[pl.BlockSpec((1, B // 2, 2 * M * PLEN),
                               lambda f: (f, 0, 0))],
        out_specs=pl.BlockSpec((1, M * PLEN, B), lambda f: (f, 0, 0)),
        out_shape=jax.ShapeDtypeStruct((F, M * PLEN, B), jnp.float32),
    )(out.reshape(F, B // 2, 2 * M * PLEN))
    return jnp.transpose(t, (2, 0, 1))
